# Initial kernel scaffold; baseline (speedup 1.0000x reference)
import functools

import jax
import jax.numpy as jnp
import numpy as np
from jax.experimental import pallas as pl
from jax.experimental.pallas import tpu as pltpu

N = 10000
E = 320000
DFEAT = 128
DG = 32
DE = 16
D = 128
H = 8
DH = D // H
L = 3

NP1 = N + 1
BLK = 128
NPAD = ((NP1 + BLK - 1) // BLK) * BLK


def _ln(x, s, b):
    m = x.mean(-1, keepdims=True)
    v = x.var(-1, keepdims=True)
    return (x - m) / jnp.sqrt(v + 1e-6) * s + b


def _encoder_body(nf_ref, w1_ref, b1_ref, lns_ref, lnb_ref, w2_ref, b2_ref,
                  out_ref):
    x = nf_ref[...]
    h = jnp.dot(x, w1_ref[...], preferred_element_type=jnp.float32) + b1_ref[...]
    m = h.mean(-1, keepdims=True)
    v = jnp.mean((h - m) * (h - m), axis=-1, keepdims=True)
    h = (h - m) * jax.lax.rsqrt(v + 1e-6) * lns_ref[...] + lnb_ref[...]
    h = jnp.maximum(h, 0.0)
    out_ref[...] = jnp.dot(h, w2_ref[...], preferred_element_type=jnp.float32) + b2_ref[...]


def _encoder(nf_pad, p):
    grid = NPAD // BLK
    return pl.pallas_call(
        _encoder_body,
        grid=(grid,),
        in_specs=[
            pl.BlockSpec((BLK, DFEAT + DG), lambda i: (i, 0)),
            pl.BlockSpec((DFEAT + DG, D), lambda i: (0, 0)),
            pl.BlockSpec((D,), lambda i: (0,)),
            pl.BlockSpec((D,), lambda i: (0,)),
            pl.BlockSpec((D,), lambda i: (0,)),
            pl.BlockSpec((D, D), lambda i: (0, 0)),
            pl.BlockSpec((D,), lambda i: (0,)),
        ],
        out_specs=pl.BlockSpec((BLK, D), lambda i: (i, 0)),
        out_shape=jax.ShapeDtypeStruct((NPAD, D), jnp.float32),
    )(nf_pad, p['enc_W1'], p['enc_b1'], p['enc_ln_s'], p['enc_ln_b'],
      p['enc_W2'], p['enc_b2'])


def _attention_block(query, keyx, key_mask, ap):
    M = keyx.shape[0]
    q = (query @ ap['Wq'] + ap['bq']).reshape(-1, H, DH)
    k = (keyx @ ap['Wk'] + ap['bk']).reshape(M, H, DH)
    v = (keyx @ ap['Wv'] + ap['bv']).reshape(M, H, DH)
    q = _ln(q, ap['qln_s'], ap['qln_b'])
    k = _ln(k, ap['kln_s'], ap['kln_b'])
    scores = jnp.einsum('qhd,khd->hqk', q, k) / np.sqrt(DH)
    scores = jnp.where((key_mask > 0)[None, None, :], scores, -1e9)
    attn = jax.nn.softmax(scores, axis=-1)
    out = jnp.einsum('hqk,khd->qhd', attn, v).reshape(-1, D)
    out = out @ ap['Wo'] + ap['bo']
    return query + out


def _gatv2(nodes, edge_features, senders, receivers, gp):
    Np = nodes.shape[0]
    hs_all = (nodes @ gp['Wl'] + gp['bl']).reshape(Np, H, DH)
    hr_all = (nodes @ gp['Wr'] + gp['br']).reshape(Np, H, DH)
    hs = hs_all[senders]
    hr = hr_all[receivers]
    he = (edge_features @ gp['We'] + gp['be']).reshape(-1, H, DH)
    m = jax.nn.leaky_relu(hs + hr + he, 0.2)
    logits = jnp.einsum('ehd,hd->eh', m, gp['att'])
    ex = jnp.exp(logits)
    num = jnp.zeros((Np, H, DH), jnp.float32).at[receivers].add(ex[..., None] * hs)
    den = jnp.zeros((Np, H), jnp.float32).at[receivers].add(ex)
    out = num / (den[..., None] + 1e-9)
    return out.reshape(Np, D)


def kernel(node_features, node_mask, edge_features, global_features, edge_list,
           edge_mask, params):
    p = params
    senders = edge_list[:, 0]
    receivers = edge_list[:, 1]
    n = node_features.shape[0]
    nf = jnp.concatenate([node_features, jnp.repeat(global_features, n, axis=0)],
                         axis=-1)
    nf = jnp.concatenate([nf, jnp.zeros((1, nf.shape[-1]), jnp.float32)], axis=0)
    nm = jnp.concatenate([node_mask, jnp.zeros((1,), jnp.float32)], axis=0)
    senders = jnp.where(edge_mask, senders, -1)
    receivers = jnp.where(edge_mask, receivers, -1)
    g = jnp.tile(p['global'], (1, 1))

    nf_pad = jnp.pad(nf, ((0, NPAD - NP1), (0, 0)))
    nodes = _encoder(nf_pad, p)[:NP1]

    g = _attention_block(g, nodes, nm, p['attn1'])
    npad = nodes.shape[0]
    nodes = jnp.concatenate([nodes, jnp.repeat(g, npad, axis=0)], axis=-1)
    nodes = jax.nn.relu(nodes @ p['mix_W'] + p['mix_b'])
    for lp in p['layers']:
        nodes = _ln(nodes, lp['ln_s'], lp['ln_b'])
        skip = nodes @ lp['skip_W'] + lp['skip_b']
        nodes = jax.nn.relu(_gatv2(nodes, edge_features, senders, receivers,
                                   lp['gat']) + skip)
    g = _attention_block(g, nodes, nm, p['attn2'])
    g = jnp.relu(_ln(g, p['final_ln_s'], p['final_ln_b'])) if False else jax.nn.relu(_ln(g, p['final_ln_s'], p['final_ln_b']))
    return g.reshape(-1)


# speedup vs baseline: 1.0616x; 1.0616x over previous
import functools

import jax
import jax.numpy as jnp
import numpy as np
from jax.experimental import pallas as pl
from jax.experimental.pallas import tpu as pltpu

N = 10000
E = 320000
DFEAT = 128
DG = 32
DE = 16
D = 128
H = 8
DH = D // H
L = 3

NP1 = N + 1
BLK = 128
NPAD = ((NP1 + BLK - 1) // BLK) * BLK


def _ln(x, s, b):
    m = x.mean(-1, keepdims=True)
    v = x.var(-1, keepdims=True)
    return (x - m) / jnp.sqrt(v + 1e-6) * s + b


def _encoder_body(nf_ref, w1_ref, b1_ref, lns_ref, lnb_ref, w2_ref, b2_ref,
                  out_ref):
    x = nf_ref[...]
    h = jnp.dot(x, w1_ref[...], preferred_element_type=jnp.float32) + b1_ref[...]
    m = h.mean(-1, keepdims=True)
    v = jnp.mean((h - m) * (h - m), axis=-1, keepdims=True)
    h = (h - m) * jax.lax.rsqrt(v + 1e-6) * lns_ref[...] + lnb_ref[...]
    h = jnp.maximum(h, 0.0)
    out_ref[...] = jnp.dot(h, w2_ref[...], preferred_element_type=jnp.float32) + b2_ref[...]


def _encoder(nf_pad, p):
    grid = NPAD // BLK
    return pl.pallas_call(
        _encoder_body,
        grid=(grid,),
        in_specs=[
            pl.BlockSpec((BLK, DFEAT + DG), lambda i: (i, 0)),
            pl.BlockSpec((DFEAT + DG, D), lambda i: (0, 0)),
            pl.BlockSpec((D,), lambda i: (0,)),
            pl.BlockSpec((D,), lambda i: (0,)),
            pl.BlockSpec((D,), lambda i: (0,)),
            pl.BlockSpec((D, D), lambda i: (0, 0)),
            pl.BlockSpec((D,), lambda i: (0,)),
        ],
        out_specs=pl.BlockSpec((BLK, D), lambda i: (i, 0)),
        out_shape=jax.ShapeDtypeStruct((NPAD, D), jnp.float32),
    )(nf_pad, p['enc_W1'], p['enc_b1'], p['enc_ln_s'], p['enc_ln_b'],
      p['enc_W2'], p['enc_b2'])


def _attention_block(query, keyx, key_mask, ap):
    M = keyx.shape[0]
    q = (query @ ap['Wq'] + ap['bq']).reshape(-1, H, DH)
    k = (keyx @ ap['Wk'] + ap['bk']).reshape(M, H, DH)
    v = (keyx @ ap['Wv'] + ap['bv']).reshape(M, H, DH)
    q = _ln(q, ap['qln_s'], ap['qln_b'])
    k = _ln(k, ap['kln_s'], ap['kln_b'])
    scores = jnp.einsum('qhd,khd->hqk', q, k) / np.sqrt(DH)
    scores = jnp.where((key_mask > 0)[None, None, :], scores, -1e9)
    attn = jax.nn.softmax(scores, axis=-1)
    out = jnp.einsum('hqk,khd->qhd', attn, v).reshape(-1, D)
    out = out @ ap['Wo'] + ap['bo']
    return query + out


def _gatv2(nodes, edge_features, senders, receivers, gp):
    Np = nodes.shape[0]
    hs_all = (nodes @ gp['Wl'] + gp['bl']).reshape(Np, H, DH)
    hr_all = (nodes @ gp['Wr'] + gp['br']).reshape(Np, H, DH)
    hs = hs_all[senders]
    hr = hr_all[receivers]
    he = (edge_features @ gp['We'] + gp['be']).reshape(-1, H, DH)
    m = jax.nn.leaky_relu(hs + hr + he, 0.2)
    logits = jnp.einsum('ehd,hd->eh', m, gp['att'])
    ex = jnp.exp(logits)
    num = jnp.zeros((Np, H, DH), jnp.float32).at[receivers].add(ex[..., None] * hs)
    den = jnp.zeros((Np, H), jnp.float32).at[receivers].add(ex)
    out = num / (den[..., None] + 1e-9)
    return out.reshape(Np, D)


def kernel(node_features, node_mask, edge_features, global_features, edge_list,
           edge_mask, params):
    p = params
    senders = edge_list[:, 0]
    receivers = edge_list[:, 1]
    n = node_features.shape[0]
    nf = jnp.concatenate([node_features, jnp.repeat(global_features, n, axis=0)],
                         axis=-1)
    nf = jnp.concatenate([nf, jnp.zeros((1, nf.shape[-1]), jnp.float32)], axis=0)
    nm = jnp.concatenate([node_mask, jnp.zeros((1,), jnp.float32)], axis=0)
    senders = jnp.where(edge_mask, senders, -1)
    receivers = jnp.where(edge_mask, receivers, -1)
    g = jnp.tile(p['global'], (1, 1))

    nf_pad = jnp.pad(nf, ((0, NPAD - NP1), (0, 0)))
    nodes = _encoder(nf_pad, p)[:NP1]

    g = _attention_block(g, nodes, nm, p['attn1'])
    npad = nodes.shape[0]
    nodes = jnp.concatenate([nodes, jnp.repeat(g, npad, axis=0)], axis=-1)
    nodes = jax.nn.relu(nodes @ p['mix_W'] + p['mix_b'])
    for lp in p['layers']:
        nodes = _ln(nodes, lp['ln_s'], lp['ln_b'])
        skip = nodes @ lp['skip_W'] + lp['skip_b']
        nodes = jax.nn.relu(_gatv2(nodes, edge_features, senders, receivers,
                                   lp['gat']) + skip)
    g = _attention_block(g, nodes, nm, p['attn2'])
    g = jax.nn.relu(_ln(g, p['final_ln_s'], p['final_ln_b']))
    return g.reshape(-1)


# speedup vs baseline: 29.6471x; 27.9260x over previous
import functools

import jax
import jax.numpy as jnp
import numpy as np
from jax import lax
from jax.experimental import pallas as pl
from jax.experimental.pallas import tpu as pltpu
from jax.experimental.pallas import tpu_sc as plsc

N = 10000
E = 320000
DFEAT = 128
DG = 32
DE = 16
D = 128
H = 8
DH = D // H
L = 3

NP1 = N + 1
BLK = 128
NPAD = ((NP1 + BLK - 1) // BLK) * BLK

NC = 2
NS = 16
NW = NC * NS
NTAB = 10112
RPT = NTAB // NS
EPW = E // NW
K = 80
DNR = NTAB // 16


def _ln(x, s, b):
    m = x.mean(-1, keepdims=True)
    v = x.var(-1, keepdims=True)
    return (x - m) / jnp.sqrt(v + 1e-6) * s + b


def _encoder_body(nf_ref, w1_ref, b1_ref, lns_ref, lnb_ref, w2_ref, b2_ref,
                  out_ref):
    x = nf_ref[...]
    h = jnp.dot(x, w1_ref[...], preferred_element_type=jnp.float32) + b1_ref[...]
    m = h.mean(-1, keepdims=True)
    v = jnp.mean((h - m) * (h - m), axis=-1, keepdims=True)
    h = (h - m) * jax.lax.rsqrt(v + 1e-6) * lns_ref[...] + lnb_ref[...]
    h = jnp.maximum(h, 0.0)
    out_ref[...] = jnp.dot(h, w2_ref[...], preferred_element_type=jnp.float32) + b2_ref[...]


def _encoder(nf_pad, p):
    grid = NPAD // BLK
    return pl.pallas_call(
        _encoder_body,
        grid=(grid,),
        in_specs=[
            pl.BlockSpec((BLK, DFEAT + DG), lambda i: (i, 0)),
            pl.BlockSpec((DFEAT + DG, D), lambda i: (0, 0)),
            pl.BlockSpec((D,), lambda i: (0,)),
            pl.BlockSpec((D,), lambda i: (0,)),
            pl.BlockSpec((D,), lambda i: (0,)),
            pl.BlockSpec((D, D), lambda i: (0, 0)),
            pl.BlockSpec((D,), lambda i: (0,)),
        ],
        out_specs=pl.BlockSpec((BLK, D), lambda i: (i, 0)),
        out_shape=jax.ShapeDtypeStruct((NPAD, D), jnp.float32),
    )(nf_pad, p['enc_W1'], p['enc_b1'], p['enc_ln_s'], p['enc_ln_b'],
      p['enc_W2'], p['enc_b2'])


def _sc_edge_body(hs_tab, hr_tab, he_hbm, snd, rcv,
                  out_num, out_den,
                  acc, den_acc, att_v, idx_s, idx_r, idx_r2,
                  hs_rows, hr_rows, he_rows, exbuf, staged_den, zbuf,
                  sem_s, sem_r, sem_e):
    cid = lax.axis_index("c")
    sid = lax.axis_index("s")
    wid = cid * NS + sid
    zero16 = jnp.zeros((16,), jnp.float32)
    iota16 = lax.iota(jnp.int32, 16)

    def zrow(r, carry):
        for c in range(8):
            zbuf[r, pl.ds(c * 16, 16)] = zero16
        return carry
    lax.fori_loop(0, 8, zrow, 0)

    def zacc(t, carry):
        pltpu.sync_copy(zbuf, acc.at[pl.ds(sid * RPT + t * 8, 8)])
        return carry
    lax.fori_loop(0, RPT // 8, zacc, 0)
    dstart = jnp.minimum(sid * 40, DNR - 40)
    for t in range(5):
        pltpu.sync_copy(zbuf, den_acc.at[pl.ds(dstart + t * 8, 8)])

    def zsd(r, carry):
        for c in range(8):
            staged_den[r, pl.ds(c * 16, 16)] = zero16
        return carry
    lax.fori_loop(0, K, zsd, 0)
    pltpu.sync_copy(hs_tab.at[pl.ds(NTAB, 1)], att_v)
    plsc.subcore_barrier()

    ebase = wid * EPW

    def batch(b, carry):
        off = ebase + b * K
        pltpu.sync_copy(snd.at[pl.ds(off, K)], idx_s)
        pltpu.sync_copy(rcv.at[pl.ds(off, K)], idx_r)
        cs = pltpu.async_copy(hs_tab.at[idx_s], hs_rows, sem_s)
        cr = pltpu.async_copy(hr_tab.at[idx_r], hr_rows, sem_r)
        ce = pltpu.async_copy(he_hbm.at[pl.ds(off, K)], he_rows, sem_e)
        cs.wait()
        cr.wait()
        ce.wait()

        def p1(k, c1):
            for j in range(H):
                sl = pl.ds(j * 16, 16)
                mv = hs_rows[k, sl] + hr_rows[k, sl] + he_rows[k, sl]
                he_rows[k, sl] = jnp.where(mv >= 0.0, mv, mv * 0.2)
            return c1
        lax.fori_loop(0, K, p1, 0)

        def p2(g, c2):
            rows = g * 16 + iota16
            rvec = idx_r[pl.ds(g * 16, 16)]
            idx_r2[pl.ds(g * 16, 16)] = lax.shift_right_logical(rvec, 4)
            posv = (rvec & 15) * 8
            for h in range(H):
                attv = att_v[0, pl.ds(h * 16, 16)]
                lg = jnp.zeros((16,), jnp.float32)
                for dd in range(16):
                    col = plsc.load_gather(
                        he_rows, [rows, jnp.full((16,), h * 16 + dd, jnp.int32)])
                    lg = lg + col * attv[dd]
                exh = jnp.exp(lg)
                plsc.store_scatter(exbuf, [g * 256 + iota16 * 16 + h], exh)
                plsc.store_scatter(staged_den, [rows, posv + h], exh)
            return c2
        lax.fori_loop(0, K // 16, p2, 0)

        def p3(k, c3):
            exrow = exbuf[pl.ds(k * 16, 16)]
            for j in range(H):
                sl = pl.ds(j * 16, 16)
                hr_rows[k, sl] = hs_rows[k, sl] * exrow[j]
            return c3
        lax.fori_loop(0, K, p3, 0)

        pltpu.sync_copy(hr_rows, acc.at[idx_r], add=True)
        pltpu.sync_copy(staged_den, den_acc.at[idx_r2], add=True)

        def pc(g, c4):
            rows = g * 16 + iota16
            posv = (idx_r[pl.ds(g * 16, 16)] & 15) * 8
            for h in range(H):
                plsc.store_scatter(staged_den, [rows, posv + h], zero16)
            return c4
        lax.fori_loop(0, K // 16, pc, 0)
        return carry
    lax.fori_loop(0, EPW // K, batch, 0)

    plsc.subcore_barrier()
    pltpu.sync_copy(acc.at[pl.ds(sid * RPT, RPT)],
                    out_num.at[cid, pl.ds(sid * RPT, RPT)])
    dstart2 = jnp.minimum(sid * 40, DNR - 40)
    pltpu.sync_copy(den_acc.at[pl.ds(dstart2, 40)],
                    out_den.at[cid, pl.ds(dstart2, 40)])


_sc_edges = pl.kernel(
    _sc_edge_body,
    out_type=(jax.ShapeDtypeStruct((NC, NTAB, D), jnp.float32),
              jax.ShapeDtypeStruct((NC, DNR, D), jnp.float32)),
    mesh=plsc.VectorSubcoreMesh(core_axis_name="c", subcore_axis_name="s",
                                num_cores=NC, num_subcores=NS),
    scratch_types=[
        pltpu.VMEM_SHARED((NTAB, D), jnp.float32),
        pltpu.VMEM_SHARED((DNR, D), jnp.float32),
        pltpu.VMEM((1, D), jnp.float32),
        pltpu.VMEM((K,), jnp.int32),
        pltpu.VMEM((K,), jnp.int32),
        pltpu.VMEM((K,), jnp.int32),
        pltpu.VMEM((K, D), jnp.float32),
        pltpu.VMEM((K, D), jnp.float32),
        pltpu.VMEM((K, D), jnp.float32),
        pltpu.VMEM((K * 16,), jnp.float32),
        pltpu.VMEM((K, D), jnp.float32),
        pltpu.VMEM((8, D), jnp.float32),
        pltpu.SemaphoreType.DMA,
        pltpu.SemaphoreType.DMA,
        pltpu.SemaphoreType.DMA,
    ],
    compiler_params=pltpu.CompilerParams(needs_layout_passes=False),
    name="sc_gatv2_edges",
)


def _gatv2_sc(nodes, edge_features, el_sc, gp):
    hs_all = nodes @ gp['Wl'] + gp['bl']
    hr_all = nodes @ gp['Wr'] + gp['br']
    he = edge_features @ gp['We'] + gp['be']
    pad = NTAB - NP1
    hs_tab = jnp.pad(hs_all, ((0, pad), (0, 0)))
    hr_tab = jnp.pad(hr_all, ((0, pad), (0, 0)))
    att_flat = gp['att'].reshape(1, D)
    hs_tab = jnp.concatenate([hs_tab, att_flat,
                              jnp.zeros((7, D), jnp.float32)], axis=0)
    num, den_t = _sc_edges(hs_tab, hr_tab, he, el_sc[0], el_sc[1])
    num = num[0, :NP1] + num[1, :NP1]
    den = (den_t[0] + den_t[1]).reshape(NTAB, H)[:NP1]
    out = num.reshape(NP1, H, DH) / (den[..., None] + 1e-9)
    return out.reshape(NP1, D)


def _attention_block(query, keyx, key_mask, ap):
    M = keyx.shape[0]
    q = (query @ ap['Wq'] + ap['bq']).reshape(-1, H, DH)
    k = (keyx @ ap['Wk'] + ap['bk']).reshape(M, H, DH)
    v = (keyx @ ap['Wv'] + ap['bv']).reshape(M, H, DH)
    q = _ln(q, ap['qln_s'], ap['qln_b'])
    k = _ln(k, ap['kln_s'], ap['kln_b'])
    scores = jnp.einsum('qhd,khd->hqk', q, k) / np.sqrt(DH)
    scores = jnp.where((key_mask > 0)[None, None, :], scores, -1e9)
    attn = jax.nn.softmax(scores, axis=-1)
    out = jnp.einsum('hqk,khd->qhd', attn, v).reshape(-1, D)
    out = out @ ap['Wo'] + ap['bo']
    return query + out


def kernel(node_features, node_mask, edge_features, global_features, edge_list,
           edge_mask, params):
    p = params
    senders = edge_list[:, 0]
    receivers = edge_list[:, 1]
    n = node_features.shape[0]
    nf = jnp.concatenate([node_features, jnp.repeat(global_features, n, axis=0)],
                         axis=-1)
    nf = jnp.concatenate([nf, jnp.zeros((1, nf.shape[-1]), jnp.float32)], axis=0)
    nm = jnp.concatenate([node_mask, jnp.zeros((1,), jnp.float32)], axis=0)
    snd_sc = jnp.where(edge_mask, senders, NTAB - 1).astype(jnp.int32)
    rcv_sc = jnp.where(edge_mask, receivers, NTAB - 1).astype(jnp.int32)
    el_sc = jnp.stack([snd_sc, rcv_sc])
    g = jnp.tile(p['global'], (1, 1))

    nf_pad = jnp.pad(nf, ((0, NPAD - NP1), (0, 0)))
    nodes = _encoder(nf_pad, p)[:NP1]

    g = _attention_block(g, nodes, nm, p['attn1'])
    npad = nodes.shape[0]
    nodes = jnp.concatenate([nodes, jnp.repeat(g, npad, axis=0)], axis=-1)
    nodes = jax.nn.relu(nodes @ p['mix_W'] + p['mix_b'])
    for lp in p['layers']:
        nodes = _ln(nodes, lp['ln_s'], lp['ln_b'])
        skip = nodes @ lp['skip_W'] + lp['skip_b']
        gat = _gatv2_sc(nodes, edge_features, el_sc, lp['gat'])
        nodes = jax.nn.relu(gat + skip)
    g = _attention_block(g, nodes, nm, p['attn2'])
    g = jax.nn.relu(_ln(g, p['final_ln_s'], p['final_ln_b']))
    return g.reshape(-1)
